# Initial kernel scaffold; baseline (speedup 1.0000x reference)
import jax
import jax.numpy as jnp
from jax.experimental import pallas as pl


def kernel(user_emb, item_emb, edge_index, edge_weight):
    raise NotImplementedError("write your pallas kernel here")


# speedup vs baseline: 2.4315x; 2.4315x over previous
import functools

import jax
import jax.numpy as jnp
from jax import lax
from jax.experimental import pallas as pl
from jax.experimental.pallas import tpu as pltpu
from jax.experimental.pallas import tpu_sc as plsc

_N_USERS = 25000
_HALF = 25000
_N = 50000
_D = 64
_E = 800000
_R128 = 6272
_EP = _R128 * 128
_TILES = 16
_RPT = _R128 // _TILES
_CH_ROWS = 2
_CHUNK = _CH_ROWS * 128
_NCHUNK = _RPT // _CH_ROWS
_ACC_ROWS = 25088
_ZROWS = _ACC_ROWS // _TILES
_DROWS = 1560


def _layer_body(table, col2, rowf, wf, out, colb, rowb, wb, lidx, rows, acc, gsem):
    c = lax.axis_index("c")
    s = lax.axis_index("s")
    base = c * _HALF

    zero16 = jnp.zeros((16,), jnp.float32)

    @pl.loop(0, _CHUNK)
    def _zrow(r):
        for d in range(4):
            rows[r, pl.ds(d * 16, 16)] = zero16

    z0 = s * _ZROWS

    @pl.loop(0, 6)
    def _zdma(i):
        pltpu.sync_copy(rows.at[pl.ds(0, _CHUNK)],
                        acc.at[pl.ds(z0 + i * _CHUNK, _CHUNK)])

    pltpu.sync_copy(rows.at[pl.ds(0, 32)], acc.at[pl.ds(z0 + 6 * _CHUNK, 32)])
    plsc.subcore_barrier()

    @pl.loop(0, _NCHUNK)
    def _chunk(k):
        r0 = s * _RPT + k * _CH_ROWS
        e0 = r0 * 128
        pltpu.sync_copy(col2.at[pl.ds(r0, _CH_ROWS)], colb)
        pltpu.sync_copy(rowf.at[pl.ds(e0, _CHUNK)], rowb)
        pltpu.sync_copy(wf.at[pl.ds(e0, _CHUNK)], wb)
        descs = [
            pltpu.async_copy(table.at[colb.at[j]],
                             rows.at[pl.ds(j * 128, 128)], gsem)
            for j in range(_CH_ROWS)
        ]
        for q in range(_CHUNK // 16):
            rv = rowb[pl.ds(q * 16, 16)]
            loc = rv - base
            ok = (loc >= 0) & (loc < _HALF)
            lidx[q // 8, pl.ds((q % 8) * 16, 16)] = jnp.where(ok, loc, _HALF)
        for dsc in descs:
            dsc.wait()

        @pl.loop(0, _CHUNK // 16)
        def _grp(g):
            eb = g * 16
            w16 = wb[pl.ds(eb, 16)]
            for t in range(16):
                bw = lax.gather(
                    w16, jnp.full((16, 1), t, jnp.int32),
                    lax.GatherDimensionNumbers(offset_dims=(),
                                               collapsed_slice_dims=(0,),
                                               start_index_map=(0,)),
                    slice_sizes=(1,),
                    mode=lax.GatherScatterMode.PROMISE_IN_BOUNDS)
                for d in range(4):
                    sl = pl.ds(d * 16, 16)
                    rows[eb + t, sl] = rows[eb + t, sl] * bw

        for j in range(_CH_ROWS):
            pltpu.sync_copy(rows.at[pl.ds(j * 128, 128)],
                            acc.at[lidx.at[j]], add=True)

    plsc.subcore_barrier()

    @pl.when(s < _TILES - 1)
    def _drain_full():
        pltpu.sync_copy(acc.at[pl.ds(s * _DROWS, _DROWS)],
                        out.at[pl.ds(base + s * _DROWS, _DROWS)])

    @pl.when(s == _TILES - 1)
    def _drain_last():
        n_last = _HALF - (_TILES - 1) * _DROWS
        pltpu.sync_copy(acc.at[pl.ds((_TILES - 1) * _DROWS, n_last)],
                        out.at[pl.ds(base + (_TILES - 1) * _DROWS, n_last)])


_layer = functools.partial(
    pl.kernel,
    out_type=jax.ShapeDtypeStruct((_N, _D), jnp.float32),
    mesh=plsc.VectorSubcoreMesh(core_axis_name="c", subcore_axis_name="s",
                                num_cores=2, num_subcores=16),
    compiler_params=pltpu.CompilerParams(use_tc_tiling_on_sc=False),
    scratch_types=[
        pltpu.VMEM((_CH_ROWS, 128), jnp.int32),
        pltpu.VMEM((_CHUNK,), jnp.int32),
        pltpu.VMEM((_CHUNK,), jnp.float32),
        pltpu.VMEM((_CH_ROWS, 128), jnp.int32),
        pltpu.VMEM((_CHUNK, _D), jnp.float32),
        pltpu.VMEM_SHARED((_ACC_ROWS, _D), jnp.float32),
        pltpu.SemaphoreType.DMA,
    ],
)(_layer_body)


def _mean_body(a, b, c, d, o):
    o[...] = (a[...] + b[...] + c[...] + d[...]) * 0.25


def _mean4(a, b, c, d):
    bs = pl.BlockSpec((1000, _D), lambda i: (i, 0))
    return pl.pallas_call(
        _mean_body,
        grid=(_N // 1000,),
        in_specs=[bs] * 4,
        out_specs=bs,
        out_shape=jax.ShapeDtypeStruct((_N, _D), jnp.float32),
    )(a, b, c, d)


def kernel(user_emb, item_emb, edge_index, edge_weight):
    ei = edge_index.astype(jnp.int32)
    row = jnp.pad(ei[0], (0, _EP - _E))
    col = jnp.pad(ei[1], (0, _EP - _E))
    w = jnp.pad(edge_weight.astype(jnp.float32), (0, _EP - _E))
    col2 = col.reshape(_R128, 128)
    table0 = jnp.concatenate([user_emb, item_emb], axis=0)
    e1 = _layer(table0, col2, row, w)
    e2 = _layer(e1, col2, row, w)
    e3 = _layer(e2, col2, row, w)
    final = _mean4(table0, e1, e2, e3)
    return final[:_N_USERS], final[_N_USERS:]
